# Initial kernel scaffold; baseline (speedup 1.0000x reference)
import jax
import jax.numpy as jnp
from jax.experimental import pallas as pl


def kernel(x_sequence, edge_index_sequence, W_s2d_0, b_s2d_0, W_d2s_0, b_d2s_0, W_s2d_1, b_s2d_1, W_d2s_1, b_d2s_1, W_ih, W_hh, b_ih, b_hh, W_p, b_p):
    raise NotImplementedError("write your pallas kernel here")


# speedup vs baseline: 10.4163x; 10.4163x over previous
import functools

import jax
import jax.numpy as jnp
from jax import lax
from jax.experimental import pallas as pl
from jax.experimental.pallas import tpu as pltpu
from jax.experimental.pallas import tpu_sc as plsc

N = 10000
T = 8
F = 128
H = 128
E = 160000
ALPHA = 0.5

N_PAD = 10240
NS = 16
NC = 2
CK = 128
CH = 80
PCH = 16
E_PAD = NS * CH * CK
ROWS_PER_TILE = N_PAD // NS
NB = 1024
NBLK = N_PAD // NB

_f32 = jnp.float32
_i32 = jnp.int32


def _deg_body(sidx_hbm, deg_hbm, idxv, accv, bncv, iotav, dsh):
    c = lax.axis_index("c")
    s = lax.axis_index("s")

    def _iota(k, _):
        iotav[pl.ds(k * 16, 16)] = lax.iota(_i32, 16) + k * 16
        return 0
    lax.fori_loop(0, (N_PAD // 128) // 16, _iota, 0)

    def _zb(i, _):
        bncv[i >> 3, pl.ds((i & 7) * 16, 16)] = jnp.zeros((16,), _f32)
        return 0
    lax.fori_loop(0, (N_PAD // 128) * 8, _zb, 0)

    @pl.when(s < 8)
    def _():
        pltpu.sync_copy(bncv, dsh.at[s])
    plsc.subcore_barrier()

    def _unit(u, _):
        tl = u >> 1
        d = u & 1
        tg = 2 * tl + c
        pltpu.sync_copy(sidx_hbm.at[tg, d, s], idxv)

        def _za(i, _):
            accv[i >> 3, pl.ds((i & 7) * 16, 16)] = jnp.zeros((16,), _f32)
            return 0
        lax.fori_loop(0, (N_PAD // 128) * 8, _za, 0)

        def _acc(i, _):
            iv = idxv[i >> 3, pl.ds((i & 7) * 16, 16)]
            cnt, last = plsc.scan_count(iv)
            plsc.addupdate_scatter(
                accv, [iv >> 7, iv & 127], cnt.astype(_f32), mask=last)
            return 0
        lax.fori_loop(0, (CH * CK) // 16, _acc, 0)

        pltpu.sync_copy(accv, dsh.at[u].at[iotav], add=True)
        return 0
    lax.fori_loop(0, 8, _unit, 0)

    plsc.subcore_barrier()

    @pl.when(s < 8)
    def _():
        pltpu.sync_copy(dsh.at[s], bncv)
        tg = 2 * (s >> 1) + c
        pltpu.sync_copy(bncv, deg_hbm.at[tg, s & 1])


_deg_kernel = functools.partial(
    pl.kernel,
    out_type=jax.ShapeDtypeStruct((T, 2, N_PAD // 128, 128), _f32),
    mesh=plsc.VectorSubcoreMesh(core_axis_name="c", subcore_axis_name="s",
                                num_cores=NC, num_subcores=NS),
    scratch_types=[
        pltpu.VMEM((CH, CK), _i32),
        pltpu.VMEM((N_PAD // 128, 128), _f32),
        pltpu.VMEM((N_PAD // 128, 128), _f32),
        pltpu.VMEM((N_PAD // 128,), _i32),
        pltpu.VMEM_SHARED((8, N_PAD // 128, 128), _f32),
    ],
    compiler_params=pltpu.CompilerParams(needs_layout_passes=False),
)(_deg_body)


def _gs_body(tables_hbm, gidx_hbm, sidx_hbm, raw_hbm, gbuf, gi, si, ash):
    c = lax.axis_index("c")
    s = lax.axis_index("s")

    def _unit(t, _):
        plsc.subcore_barrier()

        def _zg(i, _):
            gbuf[i >> 3, pl.ds((i & 7) * 16, 16)] = jnp.zeros((16,), _f32)
            return 0
        lax.fori_loop(0, CK * 8, _zg, 0)

        def _zc(k, _):
            pltpu.sync_copy(gbuf, ash.at[pl.ds(s * ROWS_PER_TILE + k * CK, CK)])
            return 0
        lax.fori_loop(0, ROWS_PER_TILE // CK, _zc, 0)
        plsc.subcore_barrier()

        def _piece(p, _):
            pltpu.sync_copy(gidx_hbm.at[t, c, s, pl.ds(p * PCH, PCH)], gi)
            pltpu.sync_copy(sidx_hbm.at[t, c, s, pl.ds(p * PCH, PCH)], si)

            def _chunk(j, _):
                pltpu.sync_copy(tables_hbm.at[c, t].at[gi.at[j]], gbuf)
                pltpu.sync_copy(gbuf, ash.at[si.at[j]], add=True)
                return 0
            lax.fori_loop(0, PCH, _chunk, 0)
            return 0
        lax.fori_loop(0, CH // PCH, _piece, 0)

        plsc.subcore_barrier()

        def _wo(k, _):
            base = s * ROWS_PER_TILE + k * CK
            pltpu.sync_copy(ash.at[pl.ds(base, CK)], gbuf)
            pltpu.sync_copy(gbuf, raw_hbm.at[c, t].at[pl.ds(base, CK)])
            return 0
        lax.fori_loop(0, ROWS_PER_TILE // CK, _wo, 0)
        return 0
    lax.fori_loop(0, T, _unit, 0)


_gs_kernel = functools.partial(
    pl.kernel,
    out_type=jax.ShapeDtypeStruct((2, T, N_PAD, 128), _f32),
    mesh=plsc.VectorSubcoreMesh(core_axis_name="c", subcore_axis_name="s",
                                num_cores=NC, num_subcores=NS),
    scratch_types=[
        pltpu.VMEM((CK, 128), _f32),
        pltpu.VMEM((PCH, CK), _i32),
        pltpu.VMEM((PCH, CK), _i32),
        pltpu.VMEM_SHARED((N_PAD, 128), _f32),
    ],
    compiler_params=pltpu.CompilerParams(needs_layout_passes=False),
)(_gs_body)


def _col(mat, col_idx, nrows):
    lanes = lax.broadcasted_iota(_i32, (1, 128), 1)
    m = (lanes == col_idx).astype(_f32)
    return jnp.sum(mat * m, axis=1, keepdims=True)


def _prep_body(x_ref, inv_ref, yz_ref):
    d = pl.program_id(0)
    t = pl.program_id(1)
    invc = _col(inv_ref[...], 2 * t + (1 - d), NB)
    yz_ref[0, 0] = x_ref[...] * invc


def _tc_prep(x2d, inv_tc):
    return pl.pallas_call(
        _prep_body,
        grid=(2, T, NBLK),
        in_specs=[
            pl.BlockSpec((NB, 128), lambda d, t, i: (i, t)),
            pl.BlockSpec((NB, 128), lambda d, t, i: (i, 0)),
        ],
        out_specs=pl.BlockSpec((1, 1, NB, 128), lambda d, t, i: (d, t, i, 0)),
        out_shape=jax.ShapeDtypeStruct((2, T, N_PAD, 128), _f32),
    )(x2d, inv_tc)


def _combine_body(emit_yz, raw_ref, inv_ref, w1_ref, w2_ref, b_ref, out_ref):
    t = pl.program_id(0)
    oi = _col(inv_ref[...], 2 * t, NB)
    ii = _col(inv_ref[...], 2 * t + 1, NB)
    a = raw_ref[0, 0] * oi
    b = raw_ref[1, 0] * ii
    h = jnp.dot(a, w1_ref[...], preferred_element_type=_f32)
    h = h + jnp.dot(b, w2_ref[...], preferred_element_type=_f32)
    h = jnp.maximum(h + b_ref[0:1, :], 0.0)
    if emit_yz:
        out_ref[0, 0] = h * ii
        out_ref[1, 0] = h * oi
    else:
        out_ref[...] = h


def _tc_combine(raw, inv_tc, w1s, w2s, bias, emit_yz):
    if emit_yz:
        out_spec = pl.BlockSpec((2, 1, NB, 128), lambda t, i: (0, t, i, 0))
        out_shape = jax.ShapeDtypeStruct((2, T, N_PAD, 128), _f32)
    else:
        out_spec = pl.BlockSpec((NB, 128), lambda t, i: (i, t))
        out_shape = jax.ShapeDtypeStruct((N_PAD, T * 128), _f32)
    return pl.pallas_call(
        functools.partial(_combine_body, emit_yz),
        grid=(T, NBLK),
        in_specs=[
            pl.BlockSpec((2, 1, NB, 128), lambda t, i: (0, t, i, 0)),
            pl.BlockSpec((NB, 128), lambda t, i: (i, 0)),
            pl.BlockSpec((128, 128), lambda t, i: (0, 0)),
            pl.BlockSpec((128, 128), lambda t, i: (0, 0)),
            pl.BlockSpec((8, 128), lambda t, i: (0, 0)),
        ],
        out_specs=out_spec,
        out_shape=out_shape,
    )(raw, inv_tc, w1s, w2s, bias)


def _lstm_body(g_ref, wih_ref, whh_ref, b_ref, wp_ref, bp_ref, out_ref):
    h = jnp.zeros((NB, 128), _f32)
    cc = jnp.zeros((NB, 128), _f32)
    for t in range(T):
        xt = g_ref[:, 128 * t:128 * (t + 1)]
        gates = jnp.dot(xt, wih_ref[...], preferred_element_type=_f32)
        gates = gates + jnp.dot(h, whh_ref[...], preferred_element_type=_f32)
        gates = gates + b_ref[0:1, :]
        ig = jax.nn.sigmoid(gates[:, 0:128])
        fg = jax.nn.sigmoid(gates[:, 128:256])
        gg = jnp.tanh(gates[:, 256:384])
        og = jax.nn.sigmoid(gates[:, 384:512])
        cc = fg * cc + ig * gg
        h = og * jnp.tanh(cc)
    out_ref[...] = jnp.dot(h, wp_ref[...], preferred_element_type=_f32)
    out_ref[...] = out_ref[...] + bp_ref[0:1, :]


def _tc_lstm(gnn2d, wih, whh, bias, wp, bp):
    return pl.pallas_call(
        _lstm_body,
        grid=(NBLK,),
        in_specs=[
            pl.BlockSpec((NB, T * 128), lambda i: (i, 0)),
            pl.BlockSpec((128, 512), lambda i: (0, 0)),
            pl.BlockSpec((128, 512), lambda i: (0, 0)),
            pl.BlockSpec((8, 512), lambda i: (0, 0)),
            pl.BlockSpec((128, 128), lambda i: (0, 0)),
            pl.BlockSpec((8, 128), lambda i: (0, 0)),
        ],
        out_specs=pl.BlockSpec((NB, 128), lambda i: (i, 0)),
        out_shape=jax.ShapeDtypeStruct((N_PAD, 128), _f32),
    )(gnn2d, wih, whh, bias, wp, bp)


def kernel(x_sequence, edge_index_sequence, W_s2d_0, b_s2d_0, W_d2s_0, b_d2s_0,
           W_s2d_1, b_s2d_1, W_d2s_1, b_d2s_1, W_ih, W_hh, b_ih, b_hh,
           W_p, b_p):
    row = edge_index_sequence[:, 0, :]
    col = edge_index_sequence[:, 1, :]
    padv = N + (jnp.arange(E_PAD - E, dtype=_i32) % (N_PAD - N))
    padb = jnp.broadcast_to(padv, (T, E_PAD - E))
    rowp = jnp.concatenate([row, padb], axis=1)
    colp = jnp.concatenate([col, padb], axis=1)
    gidx = jnp.stack([colp, rowp], axis=1).reshape(T, 2, NS, CH, CK)
    sidx = jnp.stack([rowp, colp], axis=1).reshape(T, 2, NS, CH, CK)

    deg = _deg_kernel(sidx).reshape(T, 2, N_PAD)
    inv = jnp.where(deg > 0, lax.rsqrt(jnp.maximum(deg, 1e-12)), 0.0)
    inv_tc = jnp.transpose(inv, (2, 0, 1)).reshape(N_PAD, T * 2)
    inv_tc = jnp.pad(inv_tc, ((0, 0), (0, 128 - 2 * T)))

    x2d = jnp.pad(x_sequence.reshape(N, T * F), ((0, N_PAD - N), (0, 0)))

    w1s0 = ALPHA * W_s2d_0.T
    w2s0 = (1.0 - ALPHA) * W_d2s_0.T
    bias0 = jnp.broadcast_to(ALPHA * b_s2d_0 + (1.0 - ALPHA) * b_d2s_0,
                             (8, 128))
    w1s1 = ALPHA * W_s2d_1.T
    w2s1 = (1.0 - ALPHA) * W_d2s_1.T
    bias1 = jnp.broadcast_to(ALPHA * b_s2d_1 + (1.0 - ALPHA) * b_d2s_1,
                             (8, 128))

    yz0 = _tc_prep(x2d, inv_tc)
    raw0 = _gs_kernel(yz0, gidx, sidx)
    yz1 = _tc_combine(raw0, inv_tc, w1s0, w2s0, bias0, emit_yz=True)

    raw1 = _gs_kernel(yz1, gidx, sidx)
    gnn2d = _tc_combine(raw1, inv_tc, w1s1, w2s1, bias1, emit_yz=False)

    lstm_bias = jnp.broadcast_to(b_ih + b_hh, (8, 512))
    bp = jnp.broadcast_to(b_p, (8, 128))
    pred = _tc_lstm(gnn2d, W_ih.T, W_hh.T, lstm_bias, W_p.T, bp)
    return pred[:N]


# speedup vs baseline: 14.6177x; 1.4033x over previous
import functools

import jax
import jax.numpy as jnp
from jax import lax
from jax.experimental import pallas as pl
from jax.experimental.pallas import tpu as pltpu
from jax.experimental.pallas import tpu_sc as plsc

N = 10000
T = 8
F = 128
H = 128
E = 160000
ALPHA = 0.5

N_PAD = 10240
NS = 16
NC = 2
CK = 128
CH = 80
PCH = 16
E_PAD = NS * CH * CK
ROWS_PER_TILE = N_PAD // NS
NB = 1024
NBLK = N_PAD // NB

_f32 = jnp.float32
_i32 = jnp.int32


def _deg_body(sidx_hbm, deg_hbm, idxv, accv, bncv, iotav, dsh):
    c = lax.axis_index("c")
    s = lax.axis_index("s")

    def _iota(k, _):
        iotav[pl.ds(k * 16, 16)] = lax.iota(_i32, 16) + k * 16
        return 0
    lax.fori_loop(0, (N_PAD // 128) // 16, _iota, 0)

    def _zb(i, _):
        bncv[i >> 3, pl.ds((i & 7) * 16, 16)] = jnp.zeros((16,), _f32)
        return 0
    lax.fori_loop(0, (N_PAD // 128) * 8, _zb, 0)

    @pl.when(s < 8)
    def _():
        pltpu.sync_copy(bncv, dsh.at[s])
    plsc.subcore_barrier()

    def _unit(u, _):
        tl = u >> 1
        d = u & 1
        tg = 2 * tl + c
        pltpu.sync_copy(sidx_hbm.at[tg, d, s], idxv)

        def _za(i, _):
            accv[i >> 3, pl.ds((i & 7) * 16, 16)] = jnp.zeros((16,), _f32)
            return 0
        lax.fori_loop(0, (N_PAD // 128) * 8, _za, 0)

        def _acc(i, _):
            iv = idxv[i >> 3, pl.ds((i & 7) * 16, 16)]
            cnt, last = plsc.scan_count(iv)
            plsc.addupdate_scatter(
                accv, [iv >> 7, iv & 127], cnt.astype(_f32), mask=last)
            return 0
        lax.fori_loop(0, (CH * CK) // 16, _acc, 0)

        pltpu.sync_copy(accv, dsh.at[u].at[iotav], add=True)
        return 0
    lax.fori_loop(0, 8, _unit, 0)

    plsc.subcore_barrier()

    @pl.when(s < 8)
    def _():
        pltpu.sync_copy(dsh.at[s], bncv)
        tg = 2 * (s >> 1) + c
        pltpu.sync_copy(bncv, deg_hbm.at[tg, s & 1])


_deg_kernel = functools.partial(
    pl.kernel,
    out_type=jax.ShapeDtypeStruct((T, 2, N_PAD // 128, 128), _f32),
    mesh=plsc.VectorSubcoreMesh(core_axis_name="c", subcore_axis_name="s",
                                num_cores=NC, num_subcores=NS),
    scratch_types=[
        pltpu.VMEM((CH, CK), _i32),
        pltpu.VMEM((N_PAD // 128, 128), _f32),
        pltpu.VMEM((N_PAD // 128, 128), _f32),
        pltpu.VMEM((N_PAD // 128,), _i32),
        pltpu.VMEM_SHARED((8, N_PAD // 128, 128), _f32),
    ],
    compiler_params=pltpu.CompilerParams(needs_layout_passes=False),
)(_deg_body)


def _gs_body(tables_hbm, gidx_hbm, sidx_hbm, raw_hbm, bufa, bufb, gi, si, ash,
             sema, semb):
    c = lax.axis_index("c")
    s = lax.axis_index("s")

    def _unit(t, _):
        tab = tables_hbm.at[c, t]
        plsc.subcore_barrier()

        def _zg(i, _):
            bufa[i >> 3, pl.ds((i & 7) * 16, 16)] = jnp.zeros((16,), _f32)
            return 0
        lax.fori_loop(0, CK * 8, _zg, 0)

        def _zc(k, _):
            pltpu.sync_copy(bufa, ash.at[pl.ds(s * ROWS_PER_TILE + k * CK, CK)])
            return 0
        lax.fori_loop(0, ROWS_PER_TILE // CK, _zc, 0)
        plsc.subcore_barrier()

        def _piece(p, _):
            pltpu.sync_copy(gidx_hbm.at[t, c, s, pl.ds(p * PCH, PCH)], gi)
            pltpu.sync_copy(sidx_hbm.at[t, c, s, pl.ds(p * PCH, PCH)], si)
            pltpu.async_copy(tab.at[gi.at[0]], bufa, sema)

            def _pair(q, _):
                j0 = 2 * q
                pltpu.async_copy(tab.at[gi.at[j0 + 1]], bufb, semb)
                pltpu.make_async_copy(tab.at[gi.at[j0]], bufa, sema).wait()
                pltpu.sync_copy(bufa, ash.at[si.at[j0]], add=True)

                @pl.when(q < PCH // 2 - 1)
                def _():
                    pltpu.async_copy(tab.at[gi.at[j0 + 2]], bufa, sema)
                pltpu.make_async_copy(tab.at[gi.at[j0 + 1]], bufb, semb).wait()
                pltpu.sync_copy(bufb, ash.at[si.at[j0 + 1]], add=True)
                return 0
            lax.fori_loop(0, PCH // 2, _pair, 0)
            return 0
        lax.fori_loop(0, CH // PCH, _piece, 0)

        plsc.subcore_barrier()

        def _wo(k, _):
            base = s * ROWS_PER_TILE + k * CK
            pltpu.sync_copy(ash.at[pl.ds(base, CK)], bufa)
            pltpu.sync_copy(bufa, raw_hbm.at[c, t].at[pl.ds(base, CK)])
            return 0
        lax.fori_loop(0, ROWS_PER_TILE // CK, _wo, 0)
        return 0
    lax.fori_loop(0, T, _unit, 0)


_gs_kernel = functools.partial(
    pl.kernel,
    out_type=jax.ShapeDtypeStruct((2, T, N_PAD, 128), _f32),
    mesh=plsc.VectorSubcoreMesh(core_axis_name="c", subcore_axis_name="s",
                                num_cores=NC, num_subcores=NS),
    scratch_types=[
        pltpu.VMEM((CK, 128), _f32),
        pltpu.VMEM((CK, 128), _f32),
        pltpu.VMEM((PCH, CK), _i32),
        pltpu.VMEM((PCH, CK), _i32),
        pltpu.VMEM_SHARED((N_PAD, 128), _f32),
        pltpu.SemaphoreType.DMA,
        pltpu.SemaphoreType.DMA,
    ],
    compiler_params=pltpu.CompilerParams(needs_layout_passes=False),
)(_gs_body)


def _col(mat, col_idx, nrows):
    lanes = lax.broadcasted_iota(_i32, (1, 128), 1)
    m = (lanes == col_idx).astype(_f32)
    return jnp.sum(mat * m, axis=1, keepdims=True)


def _prep_body(x_ref, inv_ref, yz_ref):
    d = pl.program_id(0)
    t = pl.program_id(1)
    invc = _col(inv_ref[...], 2 * t + (1 - d), NB)
    yz_ref[0, 0] = x_ref[...] * invc


def _tc_prep(x2d, inv_tc):
    return pl.pallas_call(
        _prep_body,
        grid=(2, T, NBLK),
        in_specs=[
            pl.BlockSpec((NB, 128), lambda d, t, i: (i, t)),
            pl.BlockSpec((NB, 128), lambda d, t, i: (i, 0)),
        ],
        out_specs=pl.BlockSpec((1, 1, NB, 128), lambda d, t, i: (d, t, i, 0)),
        out_shape=jax.ShapeDtypeStruct((2, T, N_PAD, 128), _f32),
    )(x2d, inv_tc)


def _combine_body(emit_yz, raw_ref, inv_ref, w1_ref, w2_ref, b_ref, out_ref):
    t = pl.program_id(0)
    oi = _col(inv_ref[...], 2 * t, NB)
    ii = _col(inv_ref[...], 2 * t + 1, NB)
    a = raw_ref[0, 0] * oi
    b = raw_ref[1, 0] * ii
    h = jnp.dot(a, w1_ref[...], preferred_element_type=_f32)
    h = h + jnp.dot(b, w2_ref[...], preferred_element_type=_f32)
    h = jnp.maximum(h + b_ref[0:1, :], 0.0)
    if emit_yz:
        out_ref[0, 0] = h * ii
        out_ref[1, 0] = h * oi
    else:
        out_ref[...] = h


def _tc_combine(raw, inv_tc, w1s, w2s, bias, emit_yz):
    if emit_yz:
        out_spec = pl.BlockSpec((2, 1, NB, 128), lambda t, i: (0, t, i, 0))
        out_shape = jax.ShapeDtypeStruct((2, T, N_PAD, 128), _f32)
    else:
        out_spec = pl.BlockSpec((NB, 128), lambda t, i: (i, t))
        out_shape = jax.ShapeDtypeStruct((N_PAD, T * 128), _f32)
    return pl.pallas_call(
        functools.partial(_combine_body, emit_yz),
        grid=(T, NBLK),
        in_specs=[
            pl.BlockSpec((2, 1, NB, 128), lambda t, i: (0, t, i, 0)),
            pl.BlockSpec((NB, 128), lambda t, i: (i, 0)),
            pl.BlockSpec((128, 128), lambda t, i: (0, 0)),
            pl.BlockSpec((128, 128), lambda t, i: (0, 0)),
            pl.BlockSpec((8, 128), lambda t, i: (0, 0)),
        ],
        out_specs=out_spec,
        out_shape=out_shape,
    )(raw, inv_tc, w1s, w2s, bias)


def _lstm_body(g_ref, wih_ref, whh_ref, b_ref, wp_ref, bp_ref, out_ref):
    h = jnp.zeros((NB, 128), _f32)
    cc = jnp.zeros((NB, 128), _f32)
    for t in range(T):
        xt = g_ref[:, 128 * t:128 * (t + 1)]
        gates = jnp.dot(xt, wih_ref[...], preferred_element_type=_f32)
        gates = gates + jnp.dot(h, whh_ref[...], preferred_element_type=_f32)
        gates = gates + b_ref[0:1, :]
        ig = jax.nn.sigmoid(gates[:, 0:128])
        fg = jax.nn.sigmoid(gates[:, 128:256])
        gg = jnp.tanh(gates[:, 256:384])
        og = jax.nn.sigmoid(gates[:, 384:512])
        cc = fg * cc + ig * gg
        h = og * jnp.tanh(cc)
    out_ref[...] = jnp.dot(h, wp_ref[...], preferred_element_type=_f32)
    out_ref[...] = out_ref[...] + bp_ref[0:1, :]


def _tc_lstm(gnn2d, wih, whh, bias, wp, bp):
    return pl.pallas_call(
        _lstm_body,
        grid=(NBLK,),
        in_specs=[
            pl.BlockSpec((NB, T * 128), lambda i: (i, 0)),
            pl.BlockSpec((128, 512), lambda i: (0, 0)),
            pl.BlockSpec((128, 512), lambda i: (0, 0)),
            pl.BlockSpec((8, 512), lambda i: (0, 0)),
            pl.BlockSpec((128, 128), lambda i: (0, 0)),
            pl.BlockSpec((8, 128), lambda i: (0, 0)),
        ],
        out_specs=pl.BlockSpec((NB, 128), lambda i: (i, 0)),
        out_shape=jax.ShapeDtypeStruct((N_PAD, 128), _f32),
    )(gnn2d, wih, whh, bias, wp, bp)


def kernel(x_sequence, edge_index_sequence, W_s2d_0, b_s2d_0, W_d2s_0, b_d2s_0,
           W_s2d_1, b_s2d_1, W_d2s_1, b_d2s_1, W_ih, W_hh, b_ih, b_hh,
           W_p, b_p):
    row = edge_index_sequence[:, 0, :]
    col = edge_index_sequence[:, 1, :]
    padv = N + (jnp.arange(E_PAD - E, dtype=_i32) % (N_PAD - N))
    padb = jnp.broadcast_to(padv, (T, E_PAD - E))
    rowp = jnp.concatenate([row, padb], axis=1)
    colp = jnp.concatenate([col, padb], axis=1)
    gidx = jnp.stack([colp, rowp], axis=1).reshape(T, 2, NS, CH, CK)
    sidx = jnp.stack([rowp, colp], axis=1).reshape(T, 2, NS, CH, CK)

    deg = _deg_kernel(sidx).reshape(T, 2, N_PAD)
    inv = jnp.where(deg > 0, lax.rsqrt(jnp.maximum(deg, 1e-12)), 0.0)
    inv_tc = jnp.transpose(inv, (2, 0, 1)).reshape(N_PAD, T * 2)
    inv_tc = jnp.pad(inv_tc, ((0, 0), (0, 128 - 2 * T)))

    x2d = jnp.pad(x_sequence.reshape(N, T * F), ((0, N_PAD - N), (0, 0)))

    w1s0 = ALPHA * W_s2d_0.T
    w2s0 = (1.0 - ALPHA) * W_d2s_0.T
    bias0 = jnp.broadcast_to(ALPHA * b_s2d_0 + (1.0 - ALPHA) * b_d2s_0,
                             (8, 128))
    w1s1 = ALPHA * W_s2d_1.T
    w2s1 = (1.0 - ALPHA) * W_d2s_1.T
    bias1 = jnp.broadcast_to(ALPHA * b_s2d_1 + (1.0 - ALPHA) * b_d2s_1,
                             (8, 128))

    yz0 = _tc_prep(x2d, inv_tc)
    raw0 = _gs_kernel(yz0, gidx, sidx)
    yz1 = _tc_combine(raw0, inv_tc, w1s0, w2s0, bias0, emit_yz=True)

    raw1 = _gs_kernel(yz1, gidx, sidx)
    gnn2d = _tc_combine(raw1, inv_tc, w1s1, w2s1, bias1, emit_yz=False)

    lstm_bias = jnp.broadcast_to(b_ih + b_hh, (8, 512))
    bp = jnp.broadcast_to(b_p, (8, 128))
    pred = _tc_lstm(gnn2d, W_ih.T, W_hh.T, lstm_bias, W_p.T, bp)
    return pred[:N]


# speedup vs baseline: 15.7804x; 1.0795x over previous
import functools

import jax
import jax.numpy as jnp
from jax import lax
from jax.experimental import pallas as pl
from jax.experimental.pallas import tpu as pltpu
from jax.experimental.pallas import tpu_sc as plsc

N = 10000
T = 8
F = 128
H = 128
E = 160000
ALPHA = 0.5

N_PAD = 10240
NS = 16
NC = 2
CK = 128
CH = 80
PCH = 16
E_PAD = NS * CH * CK
ROWS_PER_TILE = N_PAD // NS
NB = 1024
NBLK = N_PAD // NB

_f32 = jnp.float32
_i32 = jnp.int32


def _deg_body(eidx_hbm, deg_hbm, idxv, accv, bncv, iotav, dsh):
    c = lax.axis_index("c")
    s = lax.axis_index("s")

    def _iota(k, _):
        iotav[pl.ds(k * 16, 16)] = lax.iota(_i32, 16) + k * 16
        return 0
    lax.fori_loop(0, (N_PAD // 128) // 16, _iota, 0)

    def _zb(i, _):
        bncv[i >> 3, pl.ds((i & 7) * 16, 16)] = jnp.zeros((16,), _f32)
        return 0
    lax.fori_loop(0, (N_PAD // 128) * 8, _zb, 0)

    @pl.when(s < 8)
    def _():
        pltpu.sync_copy(bncv, dsh.at[s])
    plsc.subcore_barrier()

    def _unit(u, _):
        tl = u >> 1
        d = u & 1
        tg = 2 * tl + c
        pltpu.sync_copy(eidx_hbm.at[tg, d, s], idxv)

        def _za(i, _):
            accv[i >> 3, pl.ds((i & 7) * 16, 16)] = jnp.zeros((16,), _f32)
            return 0
        lax.fori_loop(0, (N_PAD // 128) * 8, _za, 0)

        def _acc(i, _):
            iv = idxv[i >> 3, pl.ds((i & 7) * 16, 16)]
            cnt, last = plsc.scan_count(iv)
            plsc.addupdate_scatter(
                accv, [iv >> 7, iv & 127], cnt.astype(_f32), mask=last)
            return 0
        lax.fori_loop(0, (CH * CK) // 16, _acc, 0)

        pltpu.sync_copy(accv, dsh.at[u].at[iotav], add=True)
        return 0
    lax.fori_loop(0, 8, _unit, 0)

    plsc.subcore_barrier()

    @pl.when(s < 8)
    def _():
        pltpu.sync_copy(dsh.at[s], bncv)
        tg = 2 * (s >> 1) + c
        pltpu.sync_copy(bncv, deg_hbm.at[tg, s & 1])


_deg_kernel = functools.partial(
    pl.kernel,
    out_type=jax.ShapeDtypeStruct((T, 2, N_PAD // 128, 128), _f32),
    mesh=plsc.VectorSubcoreMesh(core_axis_name="c", subcore_axis_name="s",
                                num_cores=NC, num_subcores=NS),
    scratch_types=[
        pltpu.VMEM((CH, CK), _i32),
        pltpu.VMEM((N_PAD // 128, 128), _f32),
        pltpu.VMEM((N_PAD // 128, 128), _f32),
        pltpu.VMEM((N_PAD // 128,), _i32),
        pltpu.VMEM_SHARED((8, N_PAD // 128, 128), _f32),
    ],
    compiler_params=pltpu.CompilerParams(needs_layout_passes=False),
)(_deg_body)


def _gs_body(tables_hbm, eidx_hbm, raw_hbm, bufa, bufb, gia, gib, sia, sib,
             ash, sema, semb, semgi, semsi):
    c = lax.axis_index("c")
    s = lax.axis_index("s")

    def _unit(t, _):
        tab = tables_hbm.at[c, t]
        plsc.subcore_barrier()

        def _zg(i, _):
            bufa[i >> 3, pl.ds((i & 7) * 16, 16)] = jnp.zeros((16,), _f32)
            return 0
        lax.fori_loop(0, CK * 8, _zg, 0)

        def _zc(k, _):
            pltpu.sync_copy(bufa, ash.at[pl.ds(s * ROWS_PER_TILE + k * CK, CK)])
            return 0
        lax.fori_loop(0, ROWS_PER_TILE // CK, _zc, 0)
        plsc.subcore_barrier()

        gsl = eidx_hbm.at[t, 1 - c, s]
        ssl = eidx_hbm.at[t, c, s]
        pltpu.sync_copy(gsl.at[pl.ds(0, PCH)], gia)
        pltpu.sync_copy(ssl.at[pl.ds(0, PCH)], sia)
        pltpu.async_copy(tab.at[gia.at[0]], bufa, sema)

        for p in range(CH // PCH):
            gi_c, si_c = (gia, sia) if p % 2 == 0 else (gib, sib)
            gi_n, si_n = (gib, sib) if p % 2 == 0 else (gia, sia)
            last = p == CH // PCH - 1
            if not last:
                nsl = pl.ds((p + 1) * PCH, PCH)
                pltpu.async_copy(gsl.at[nsl], gi_n, semgi)
                pltpu.async_copy(ssl.at[nsl], si_n, semsi)

            def _pair(q, _, gi_c=gi_c, si_c=si_c):
                j0 = 2 * q
                pltpu.async_copy(tab.at[gi_c.at[j0 + 1]], bufb, semb)
                pltpu.make_async_copy(tab.at[gi_c.at[j0]], bufa, sema).wait()
                pltpu.sync_copy(bufa, ash.at[si_c.at[j0]], add=True)
                pltpu.async_copy(tab.at[gi_c.at[j0 + 2]], bufa, sema)
                pltpu.make_async_copy(
                    tab.at[gi_c.at[j0 + 1]], bufb, semb).wait()
                pltpu.sync_copy(bufb, ash.at[si_c.at[j0 + 1]], add=True)
                return 0
            lax.fori_loop(0, PCH // 2 - 1, _pair, 0)

            pltpu.async_copy(tab.at[gi_c.at[PCH - 1]], bufb, semb)
            if not last:
                pltpu.make_async_copy(gsl.at[nsl], gi_n, semgi).wait()
                pltpu.make_async_copy(ssl.at[nsl], si_n, semsi).wait()
            pltpu.make_async_copy(tab.at[gi_c.at[PCH - 2]], bufa, sema).wait()
            pltpu.sync_copy(bufa, ash.at[si_c.at[PCH - 2]], add=True)
            if not last:
                pltpu.async_copy(tab.at[gi_n.at[0]], bufa, sema)
            pltpu.make_async_copy(tab.at[gi_c.at[PCH - 1]], bufb, semb).wait()
            pltpu.sync_copy(bufb, ash.at[si_c.at[PCH - 1]], add=True)

        plsc.subcore_barrier()

        def _wo(k, _):
            base = s * ROWS_PER_TILE + k * CK
            pltpu.sync_copy(ash.at[pl.ds(base, CK)],
                            raw_hbm.at[c, t].at[pl.ds(base, CK)])
            return 0
        lax.fori_loop(0, ROWS_PER_TILE // CK, _wo, 0)
        return 0
    lax.fori_loop(0, T, _unit, 0)


_gs_kernel = functools.partial(
    pl.kernel,
    out_type=jax.ShapeDtypeStruct((2, T, N_PAD, 128), _f32),
    mesh=plsc.VectorSubcoreMesh(core_axis_name="c", subcore_axis_name="s",
                                num_cores=NC, num_subcores=NS),
    scratch_types=[
        pltpu.VMEM((CK, 128), _f32),
        pltpu.VMEM((CK, 128), _f32),
        pltpu.VMEM((PCH, CK), _i32),
        pltpu.VMEM((PCH, CK), _i32),
        pltpu.VMEM((PCH, CK), _i32),
        pltpu.VMEM((PCH, CK), _i32),
        pltpu.VMEM_SHARED((N_PAD, 128), _f32),
        pltpu.SemaphoreType.DMA,
        pltpu.SemaphoreType.DMA,
        pltpu.SemaphoreType.DMA,
        pltpu.SemaphoreType.DMA,
    ],
    compiler_params=pltpu.CompilerParams(needs_layout_passes=False),
)(_gs_body)


def _col(mat, col_idx, nrows):
    lanes = lax.broadcasted_iota(_i32, (1, 128), 1)
    m = (lanes == col_idx).astype(_f32)
    return jnp.sum(mat * m, axis=1, keepdims=True)


def _prep_body(x_ref, inv_ref, yz_ref):
    d = pl.program_id(0)
    t = pl.program_id(1)
    invc = _col(inv_ref[...], 2 * t + (1 - d), NB)
    yz_ref[0, 0] = x_ref[...] * invc


def _tc_prep(x2d, inv_tc):
    return pl.pallas_call(
        _prep_body,
        grid=(2, T, NBLK),
        in_specs=[
            pl.BlockSpec((NB, 128), lambda d, t, i: (i, t)),
            pl.BlockSpec((NB, 128), lambda d, t, i: (i, 0)),
        ],
        out_specs=pl.BlockSpec((1, 1, NB, 128), lambda d, t, i: (d, t, i, 0)),
        out_shape=jax.ShapeDtypeStruct((2, T, N_PAD, 128), _f32),
    )(x2d, inv_tc)


def _combine_body(emit_yz, raw_ref, inv_ref, w1_ref, w2_ref, b_ref, out_ref):
    t = pl.program_id(0)
    oi = _col(inv_ref[...], 2 * t, NB)
    ii = _col(inv_ref[...], 2 * t + 1, NB)
    a = raw_ref[0, 0] * oi
    b = raw_ref[1, 0] * ii
    h = jnp.dot(a, w1_ref[...], preferred_element_type=_f32)
    h = h + jnp.dot(b, w2_ref[...], preferred_element_type=_f32)
    h = jnp.maximum(h + b_ref[0:1, :], 0.0)
    if emit_yz:
        out_ref[0, 0] = h * ii
        out_ref[1, 0] = h * oi
    else:
        out_ref[...] = h


def _tc_combine(raw, inv_tc, w1s, w2s, bias, emit_yz):
    if emit_yz:
        out_spec = pl.BlockSpec((2, 1, NB, 128), lambda t, i: (0, t, i, 0))
        out_shape = jax.ShapeDtypeStruct((2, T, N_PAD, 128), _f32)
    else:
        out_spec = pl.BlockSpec((NB, 128), lambda t, i: (i, t))
        out_shape = jax.ShapeDtypeStruct((N_PAD, T * 128), _f32)
    return pl.pallas_call(
        functools.partial(_combine_body, emit_yz),
        grid=(T, NBLK),
        in_specs=[
            pl.BlockSpec((2, 1, NB, 128), lambda t, i: (0, t, i, 0)),
            pl.BlockSpec((NB, 128), lambda t, i: (i, 0)),
            pl.BlockSpec((128, 128), lambda t, i: (0, 0)),
            pl.BlockSpec((128, 128), lambda t, i: (0, 0)),
            pl.BlockSpec((8, 128), lambda t, i: (0, 0)),
        ],
        out_specs=out_spec,
        out_shape=out_shape,
    )(raw, inv_tc, w1s, w2s, bias)


def _lstm_body(g_ref, wih_ref, whh_ref, b_ref, wp_ref, bp_ref, out_ref):
    h = jnp.zeros((NB, 128), _f32)
    cc = jnp.zeros((NB, 128), _f32)
    for t in range(T):
        xt = g_ref[:, 128 * t:128 * (t + 1)]
        gates = jnp.dot(xt, wih_ref[...], preferred_element_type=_f32)
        gates = gates + jnp.dot(h, whh_ref[...], preferred_element_type=_f32)
        gates = gates + b_ref[0:1, :]
        ig = jax.nn.sigmoid(gates[:, 0:128])
        fg = jax.nn.sigmoid(gates[:, 128:256])
        gg = jnp.tanh(gates[:, 256:384])
        og = jax.nn.sigmoid(gates[:, 384:512])
        cc = fg * cc + ig * gg
        h = og * jnp.tanh(cc)
    out_ref[...] = jnp.dot(h, wp_ref[...], preferred_element_type=_f32)
    out_ref[...] = out_ref[...] + bp_ref[0:1, :]


def _tc_lstm(gnn2d, wih, whh, bias, wp, bp):
    return pl.pallas_call(
        _lstm_body,
        grid=(NBLK,),
        in_specs=[
            pl.BlockSpec((NB, T * 128), lambda i: (i, 0)),
            pl.BlockSpec((128, 512), lambda i: (0, 0)),
            pl.BlockSpec((128, 512), lambda i: (0, 0)),
            pl.BlockSpec((8, 512), lambda i: (0, 0)),
            pl.BlockSpec((128, 128), lambda i: (0, 0)),
            pl.BlockSpec((8, 128), lambda i: (0, 0)),
        ],
        out_specs=pl.BlockSpec((NB, 128), lambda i: (i, 0)),
        out_shape=jax.ShapeDtypeStruct((N_PAD, 128), _f32),
    )(gnn2d, wih, whh, bias, wp, bp)


def kernel(x_sequence, edge_index_sequence, W_s2d_0, b_s2d_0, W_d2s_0, b_d2s_0,
           W_s2d_1, b_s2d_1, W_d2s_1, b_d2s_1, W_ih, W_hh, b_ih, b_hh,
           W_p, b_p):
    padv = N + (jnp.arange(E_PAD - E, dtype=_i32) % (N_PAD - N))
    padb = jnp.broadcast_to(padv, (T, 2, E_PAD - E))
    eidx = jnp.concatenate([edge_index_sequence, padb],
                           axis=2).reshape(T, 2, NS, CH, CK)

    deg = _deg_kernel(eidx).reshape(T, 2, N_PAD)
    inv = jnp.where(deg > 0, lax.rsqrt(jnp.maximum(deg, 1e-12)), 0.0)
    inv_tc = jnp.transpose(inv, (2, 0, 1)).reshape(N_PAD, T * 2)
    inv_tc = jnp.pad(inv_tc, ((0, 0), (0, 128 - 2 * T)))

    x2d = jnp.pad(x_sequence.reshape(N, T * F), ((0, N_PAD - N), (0, 0)))

    w1s0 = ALPHA * W_s2d_0.T
    w2s0 = (1.0 - ALPHA) * W_d2s_0.T
    bias0 = jnp.broadcast_to(ALPHA * b_s2d_0 + (1.0 - ALPHA) * b_d2s_0,
                             (8, 128))
    w1s1 = ALPHA * W_s2d_1.T
    w2s1 = (1.0 - ALPHA) * W_d2s_1.T
    bias1 = jnp.broadcast_to(ALPHA * b_s2d_1 + (1.0 - ALPHA) * b_d2s_1,
                             (8, 128))

    yz0 = _tc_prep(x2d, inv_tc)
    raw0 = _gs_kernel(yz0, eidx)
    yz1 = _tc_combine(raw0, inv_tc, w1s0, w2s0, bias0, emit_yz=True)

    raw1 = _gs_kernel(yz1, eidx)
    gnn2d = _tc_combine(raw1, inv_tc, w1s1, w2s1, bias1, emit_yz=False)

    lstm_bias = jnp.broadcast_to(b_ih + b_hh, (8, 512))
    bp = jnp.broadcast_to(b_p, (8, 128))
    pred = _tc_lstm(gnn2d, W_ih.T, W_hh.T, lstm_bias, W_p.T, bp)
    return pred[:N]
